# baseline (speedup 1.0000x reference)
import jax
import jax.numpy as jnp
from jax.experimental import pallas as pl

_P = 0.1
_B, _L, _V = 16384, 20, 128
_ROWS = _B * _L

_ROW_BLK = 2048


def _main_body(f_ref, pe_ref, m_ref, o_ref):
    m = m_ref[...]
    f = f_ref[0, 0]
    pe = pe_ref[0, 0]
    col = jax.lax.broadcasted_iota(jnp.int32, (1, _V), 1)
    scale = jnp.where(col == 0, 1.0, f)
    eos = m[:, :1]
    rest_sum = jnp.sum(m, axis=1, keepdims=True) - eos
    o_ref[...] = jnp.concatenate([m * scale, pe * rest_sum], axis=1)


def _ent_body(c_ref, e_ref, sym_ref, me_ref, mn_ref):
    e = e_ref[...]
    c = c_ref[0, 0]
    sym = e + c
    sym_ref[...] = sym
    me_ref[...] = jnp.sum(sym, axis=1, keepdims=True)
    mn_ref[...] = jnp.sum(e, axis=1, keepdims=True)


def kernel(messages, apply_noise, entropy):
    p = jnp.float32(_P)
    h2 = -p * jnp.log2(p) - (1.0 - p) * jnp.log2(1.0 - p)
    an = jnp.asarray(apply_noise)
    f = jnp.where(an, 1.0 - p, 1.0).astype(jnp.float32).reshape(1, 1)
    pe = jnp.where(an, p, 0.0).astype(jnp.float32).reshape(1, 1)
    c = jnp.where(an, h2, 0.0).astype(jnp.float32).reshape(1, 1)

    mf = messages.reshape(_ROWS, _V)
    scalar_spec = pl.BlockSpec((1, 1), lambda i: (0, 0))
    out_flat = pl.pallas_call(
        _main_body,
        grid=(_ROWS // _ROW_BLK,),
        in_specs=[
            scalar_spec,
            scalar_spec,
            pl.BlockSpec((_ROW_BLK, _V), lambda i: (i, 0)),
        ],
        out_specs=pl.BlockSpec((_ROW_BLK, _V + 1), lambda i: (i, 0)),
        out_shape=jax.ShapeDtypeStruct((_ROWS, _V + 1), jnp.float32),
    )(f, pe, mf)
    out_messages = out_flat.reshape(_B, _L, _V + 1)

    eb = 4096
    sym, me, mn = pl.pallas_call(
        _ent_body,
        grid=(_B // eb,),
        in_specs=[
            scalar_spec,
            pl.BlockSpec((eb, _L), lambda i: (i, 0)),
        ],
        out_specs=[
            pl.BlockSpec((eb, _L), lambda i: (i, 0)),
            pl.BlockSpec((eb, 1), lambda i: (i, 0)),
            pl.BlockSpec((eb, 1), lambda i: (i, 0)),
        ],
        out_shape=[
            jax.ShapeDtypeStruct((_B, _L), jnp.float32),
            jax.ShapeDtypeStruct((_B, 1), jnp.float32),
            jax.ShapeDtypeStruct((_B, 1), jnp.float32),
        ],
    )(c, entropy)

    message_entropy = me.reshape(_B)
    message_nn = mn.reshape(_B)
    return (out_messages, message_entropy, sym, message_nn, entropy)


# speedup vs baseline: 1.5189x; 1.5189x over previous
import jax
import jax.numpy as jnp
from jax.experimental import pallas as pl

_P = 0.1
_B, _L, _V = 16384, 20, 128

_BB = 256


def _body(f_ref, pe_ref, c_ref, m_ref, e_ref,
          o_ref, sym_ref, me_ref, mn_ref):
    m = m_ref[...]
    f = f_ref[0, 0]
    pe = pe_ref[0, 0]
    c = c_ref[0, 0]
    col = jax.lax.broadcasted_iota(jnp.int32, (1, 1, _V), 2)
    scale = jnp.where(col == 0, 1.0, f)
    eos = m[..., :1]
    rest_sum = jnp.sum(m, axis=-1, keepdims=True) - eos
    o_ref[...] = jnp.concatenate([m * scale, pe * rest_sum], axis=-1)

    e = e_ref[...]
    sym = e + c
    sym_ref[...] = sym
    me_ref[...] = jnp.sum(sym, axis=1, keepdims=True)
    mn_ref[...] = jnp.sum(e, axis=1, keepdims=True)


def kernel(messages, apply_noise, entropy):
    p = jnp.float32(_P)
    h2 = -p * jnp.log2(p) - (1.0 - p) * jnp.log2(1.0 - p)
    an = jnp.asarray(apply_noise)
    f = jnp.where(an, 1.0 - p, 1.0).astype(jnp.float32).reshape(1, 1)
    pe = jnp.where(an, p, 0.0).astype(jnp.float32).reshape(1, 1)
    c = jnp.where(an, h2, 0.0).astype(jnp.float32).reshape(1, 1)

    scalar_spec = pl.BlockSpec((1, 1), lambda i: (0, 0))
    out, sym, me, mn = pl.pallas_call(
        _body,
        grid=(_B // _BB,),
        in_specs=[
            scalar_spec,
            scalar_spec,
            scalar_spec,
            pl.BlockSpec((_BB, _L, _V), lambda i: (i, 0, 0)),
            pl.BlockSpec((_BB, _L), lambda i: (i, 0)),
        ],
        out_specs=[
            pl.BlockSpec((_BB, _L, _V + 1), lambda i: (i, 0, 0)),
            pl.BlockSpec((_BB, _L), lambda i: (i, 0)),
            pl.BlockSpec((_BB, 1), lambda i: (i, 0)),
            pl.BlockSpec((_BB, 1), lambda i: (i, 0)),
        ],
        out_shape=[
            jax.ShapeDtypeStruct((_B, _L, _V + 1), jnp.float32),
            jax.ShapeDtypeStruct((_B, _L), jnp.float32),
            jax.ShapeDtypeStruct((_B, 1), jnp.float32),
            jax.ShapeDtypeStruct((_B, 1), jnp.float32),
        ],
    )(f, pe, c, messages, entropy)

    return (out, me.reshape(_B), sym, mn.reshape(_B), entropy)
